# Initial kernel scaffold; baseline (speedup 1.0000x reference)
import jax
import jax.numpy as jnp
from jax.experimental import pallas as pl


def kernel(features, rois):
    raise NotImplementedError("write your pallas kernel here")


# speedup vs baseline: 4.0571x; 4.0571x over previous
import functools

import jax
import jax.numpy as jnp
from jax import lax
from jax.experimental import pallas as pl
from jax.experimental.pallas import tpu as pltpu
from jax.experimental.pallas import tpu_sc as plsc

B, C, H, W = 2, 256, 128, 128
R = 1000
PH, PW = 7, 7
P = PH * PW
SPATIAL_SCALE = 0.25

NC, NS, L = 2, 16, 16
NW = NC * NS
RPW = 32
SPW = RPW * P
STRIDE = 208
NGRP = SPW // L
PA = 24
PB = P - PA
GB = 112

_mesh = plsc.VectorSubcoreMesh(core_axis_name="c", subcore_axis_name="s")


@functools.partial(
    pl.kernel,
    out_type=jax.ShapeDtypeStruct((R * C * P,), jnp.float32),
    mesh=_mesh,
    compiler_params=pltpu.CompilerParams(needs_layout_passes=False),
    scratch_types=[
        pltpu.VMEM((R * 5,), jnp.float32),
        pltpu.VMEM((RPW * STRIDE,), jnp.int32),
        pltpu.VMEM((RPW * STRIDE,), jnp.float32),
        pltpu.VMEM((PA * 4, C), jnp.float32),
        pltpu.VMEM((GB, C), jnp.float32),
        pltpu.VMEM((C * P,), jnp.float32),
        pltpu.VMEM((C * P,), jnp.float32),
        pltpu.SemaphoreType.DMA,
        pltpu.SemaphoreType.DMA,
        pltpu.SemaphoreType.DMA,
        pltpu.SemaphoreType.DMA,
    ],
)
def _roi_sample_sc(table_hbm, rois_hbm, out_hbm, rois_v, idx_v, wts_v,
                   rows_a, rows_b, ob0, ob1, sga, sgb, so0, so1):
    wid = lax.axis_index("s") * NC + lax.axis_index("c")
    rbase = wid * RPW

    pltpu.sync_copy(rois_hbm, rois_v)

    iota = lax.iota(jnp.int32, L)
    iota49 = iota * P
    zero16 = jnp.zeros((L,), jnp.int32)

    @pl.loop(0, NGRP)
    def _compute_indices(g):
        s_loc = g * L + iota
        j = s_loc // P
        p = s_loc - j * P
        r = jnp.minimum(rbase + j, R - 1)
        r5 = r * 5
        b = plsc.load_gather(rois_v, [r5]).astype(jnp.int32)
        x1 = plsc.load_gather(rois_v, [r5 + 1])
        y1 = plsc.load_gather(rois_v, [r5 + 2])
        x2 = plsc.load_gather(rois_v, [r5 + 3])
        y2 = plsc.load_gather(rois_v, [r5 + 4])
        relx = (p % PW).astype(jnp.float32) * (1.0 / PW) + (0.5 / PW)
        rely = (p // PW).astype(jnp.float32) * (1.0 / PH) + (0.5 / PH)
        px = (x1 + relx * (x2 - x1)) * SPATIAL_SCALE - 0.5
        py = (y1 + rely * (y2 - y1)) * SPATIAL_SCALE - 0.5
        x0 = (px + 1.0).astype(jnp.int32) - 1
        y0 = (py + 1.0).astype(jnp.int32) - 1
        wx1 = px - x0.astype(jnp.float32)
        wx0 = 1.0 - wx1
        wy1 = py - y0.astype(jnp.float32)
        wy0 = 1.0 - wy1
        vx0 = jnp.where(x0 >= 0, 1.0, 0.0)
        vx1 = jnp.where(x0 + 1 <= W - 1, 1.0, 0.0)
        vy0 = jnp.where(y0 >= 0, 1.0, 0.0)
        vy1 = jnp.where(y0 + 1 <= H - 1, 1.0, 0.0)
        xc0 = jnp.clip(x0, 0, W - 1)
        xc1 = jnp.clip(x0 + 1, 0, W - 1)
        yc0 = jnp.clip(y0, 0, H - 1)
        yc1 = jnp.clip(y0 + 1, 0, H - 1)
        base = b * (H * W)
        row0 = base + yc0 * W
        row1 = base + yc1 * W
        pos = j * STRIDE + p * 4
        plsc.store_scatter(idx_v, [pos], row0 + xc0)
        plsc.store_scatter(idx_v, [pos + 1], row0 + xc1)
        plsc.store_scatter(idx_v, [pos + 2], row1 + xc0)
        plsc.store_scatter(idx_v, [pos + 3], row1 + xc1)
        plsc.store_scatter(wts_v, [pos], wy0 * wx0 * vy0 * vx0)
        plsc.store_scatter(wts_v, [pos + 1], wy0 * wx1 * vy0 * vx1)
        plsc.store_scatter(wts_v, [pos + 2], wy1 * wx0 * vy1 * vx0)
        plsc.store_scatter(wts_v, [pos + 3], wy1 * wx1 * vy1 * vx1)

    @pl.loop(0, RPW)
    def _pad_indices(j):
        plsc.store_scatter(idx_v, [j * STRIDE + 196 + iota], zero16,
                           mask=iota < 12)

    def issue_a(j):
        pltpu.async_copy(
            table_hbm.at[idx_v.at[pl.ds(j * STRIDE, PA * 4)]], rows_a, sga)

    def issue_b(j):
        pltpu.async_copy(
            table_hbm.at[idx_v.at[pl.ds(j * STRIDE + PA * 4, GB)]],
            rows_b, sgb)

    def wait_a(j):
        pltpu.make_async_copy(
            table_hbm.at[idx_v.at[pl.ds(j * STRIDE, PA * 4)]], rows_a,
            sga).wait()

    def wait_b(j):
        pltpu.make_async_copy(
            table_hbm.at[idx_v.at[pl.ds(j * STRIDE + PA * 4, GB)]],
            rows_b, sgb).wait()

    def accumulate(j, rows, ob, p_lo, p_hi, row_off):
        @pl.loop(p_lo, p_hi)
        def _point(p):
            wbase_i = j * STRIDE + p * 4
            rb = p * 4 - row_off
            w0 = plsc.load_gather(wts_v, [zero16 + wbase_i])
            w1 = plsc.load_gather(wts_v, [zero16 + (wbase_i + 1)])
            w2 = plsc.load_gather(wts_v, [zero16 + (wbase_i + 2)])
            w3 = plsc.load_gather(wts_v, [zero16 + (wbase_i + 3)])
            for g in range(C // L):
                col = pl.ds(g * L, L)
                acc = rows[rb, col] * w0
                acc = acc + rows[rb + 1, col] * w1
                acc = acc + rows[rb + 2, col] * w2
                acc = acc + rows[rb + 3, col] * w3
                plsc.store_scatter(ob, [iota49 + (g * L * P + p)], acc)

    def out_slice(r_glob):
        return out_hbm.at[pl.ds(r_glob * (C * P), C * P)]

    def one_roi(j, ob, so):
        r_glob = rbase + j

        @pl.when(jnp.logical_and(j >= 2, r_glob - 2 < R))
        def _wait_prev_out():
            pltpu.make_async_copy(ob, out_slice(0), so).wait()

        wait_a(j)
        issue_b(j)
        accumulate(j, rows_a, ob, 0, PA, 0)
        wait_b(j)

        @pl.when(j + 1 < RPW)
        def _prefetch_next():
            issue_a(j + 1)

        accumulate(j, rows_b, ob, PA, P, PA * 4)

        @pl.when(r_glob < R)
        def _write_out():
            pltpu.async_copy(ob, out_slice(r_glob), so)

    issue_a(0)

    @pl.loop(0, RPW, step=2)
    def _roi_pair(j):
        one_roi(j, ob0, so0)
        one_roi(j + 1, ob1, so1)

    @pl.when(rbase + RPW - 2 < R)
    def _drain0():
        pltpu.make_async_copy(ob0, out_slice(0), so0).wait()

    @pl.when(rbase + RPW - 1 < R)
    def _drain1():
        pltpu.make_async_copy(ob1, out_slice(0), so1).wait()


def kernel(features, rois):
    table = features.transpose(0, 2, 3, 1).reshape(B * H * W, C)
    out_flat = _roi_sample_sc(table, rois.reshape(-1))
    return out_flat.reshape(R, C, PH, PW)


# speedup vs baseline: 4.0708x; 1.0034x over previous
import functools

import jax
import jax.numpy as jnp
from jax import lax
from jax.experimental import pallas as pl
from jax.experimental.pallas import tpu as pltpu
from jax.experimental.pallas import tpu_sc as plsc

B, C, H, W = 2, 256, 128, 128
R = 1000
PH, PW = 7, 7
P = PH * PW
SPATIAL_SCALE = 0.25

NC, NS, L = 2, 16, 16
NW = NC * NS
RPW = 32
SPW = RPW * P
STRIDE = 208
NGRP = SPW // L
PA = 24
PB = P - PA
GB = 112

_mesh = plsc.VectorSubcoreMesh(core_axis_name="c", subcore_axis_name="s")


@functools.partial(
    pl.kernel,
    out_type=jax.ShapeDtypeStruct((R * C * P,), jnp.float32),
    mesh=_mesh,
    compiler_params=pltpu.CompilerParams(needs_layout_passes=False),
    scratch_types=[
        pltpu.VMEM((R * 5,), jnp.float32),
        pltpu.VMEM((RPW * STRIDE,), jnp.int32),
        pltpu.VMEM((RPW * STRIDE,), jnp.float32),
        pltpu.VMEM((PA * 4, C), jnp.float32),
        pltpu.VMEM((GB, C), jnp.float32),
        pltpu.VMEM((C * P,), jnp.float32),
        pltpu.VMEM((C * P,), jnp.float32),
        pltpu.SemaphoreType.DMA,
        pltpu.SemaphoreType.DMA,
        pltpu.SemaphoreType.DMA,
        pltpu.SemaphoreType.DMA,
    ],
)
def _roi_sample_sc(table_hbm, rois_hbm, out_hbm, rois_v, idx_v, wts_v,
                   rows_a, rows_b, ob0, ob1, sga, sgb, so0, so1):
    wid = lax.axis_index("s") * NC + lax.axis_index("c")
    rbase = wid * RPW

    pltpu.sync_copy(rois_hbm, rois_v)

    iota = lax.iota(jnp.int32, L)
    iota49 = iota * P
    zero16 = jnp.zeros((L,), jnp.int32)

    @pl.loop(0, NGRP)
    def _compute_indices(g):
        s_loc = g * L + iota
        j = s_loc // P
        p = s_loc - j * P
        r = jnp.minimum(rbase + j, R - 1)
        r5 = r * 5
        b = plsc.load_gather(rois_v, [r5]).astype(jnp.int32)
        x1 = plsc.load_gather(rois_v, [r5 + 1])
        y1 = plsc.load_gather(rois_v, [r5 + 2])
        x2 = plsc.load_gather(rois_v, [r5 + 3])
        y2 = plsc.load_gather(rois_v, [r5 + 4])
        relx = (p % PW).astype(jnp.float32) * (1.0 / PW) + (0.5 / PW)
        rely = (p // PW).astype(jnp.float32) * (1.0 / PH) + (0.5 / PH)
        px = (x1 + relx * (x2 - x1)) * SPATIAL_SCALE - 0.5
        py = (y1 + rely * (y2 - y1)) * SPATIAL_SCALE - 0.5
        x0 = (px + 1.0).astype(jnp.int32) - 1
        y0 = (py + 1.0).astype(jnp.int32) - 1
        wx1 = px - x0.astype(jnp.float32)
        wx0 = 1.0 - wx1
        wy1 = py - y0.astype(jnp.float32)
        wy0 = 1.0 - wy1
        vx0 = jnp.where(x0 >= 0, 1.0, 0.0)
        vx1 = jnp.where(x0 + 1 <= W - 1, 1.0, 0.0)
        vy0 = jnp.where(y0 >= 0, 1.0, 0.0)
        vy1 = jnp.where(y0 + 1 <= H - 1, 1.0, 0.0)
        xc0 = jnp.clip(x0, 0, W - 1)
        xc1 = jnp.clip(x0 + 1, 0, W - 1)
        yc0 = jnp.clip(y0, 0, H - 1)
        yc1 = jnp.clip(y0 + 1, 0, H - 1)
        base = b * (H * W)
        row0 = base + yc0 * W
        row1 = base + yc1 * W
        pos = j * STRIDE + p * 4
        plsc.store_scatter(idx_v, [pos], row0 + xc0)
        plsc.store_scatter(idx_v, [pos + 1], row0 + xc1)
        plsc.store_scatter(idx_v, [pos + 2], row1 + xc0)
        plsc.store_scatter(idx_v, [pos + 3], row1 + xc1)
        plsc.store_scatter(wts_v, [pos], wy0 * wx0 * vy0 * vx0)
        plsc.store_scatter(wts_v, [pos + 1], wy0 * wx1 * vy0 * vx1)
        plsc.store_scatter(wts_v, [pos + 2], wy1 * wx0 * vy1 * vx0)
        plsc.store_scatter(wts_v, [pos + 3], wy1 * wx1 * vy1 * vx1)

    @pl.loop(0, RPW)
    def _pad_indices(j):
        plsc.store_scatter(idx_v, [j * STRIDE + 196 + iota], zero16,
                           mask=iota < 12)

    def issue_a(j):
        pltpu.async_copy(
            table_hbm.at[idx_v.at[pl.ds(j * STRIDE, PA * 4)]], rows_a, sga)

    def issue_b(j):
        pltpu.async_copy(
            table_hbm.at[idx_v.at[pl.ds(j * STRIDE + PA * 4, GB)]],
            rows_b, sgb)

    def wait_a(j):
        pltpu.make_async_copy(
            table_hbm.at[idx_v.at[pl.ds(j * STRIDE, PA * 4)]], rows_a,
            sga).wait()

    def wait_b(j):
        pltpu.make_async_copy(
            table_hbm.at[idx_v.at[pl.ds(j * STRIDE + PA * 4, GB)]],
            rows_b, sgb).wait()

    def accumulate(j, rows, ob, p_lo, p_hi, row_off):
        @pl.loop(p_lo, p_hi)
        def _point(p):
            wbase_i = j * STRIDE + p * 4
            rb = p * 4 - row_off
            w0 = plsc.load_gather(wts_v, [zero16 + wbase_i])
            w1 = plsc.load_gather(wts_v, [zero16 + (wbase_i + 1)])
            w2 = plsc.load_gather(wts_v, [zero16 + (wbase_i + 2)])
            w3 = plsc.load_gather(wts_v, [zero16 + (wbase_i + 3)])
            for g in range(0, C // L, 2):
                c0 = pl.ds(g * L, L)
                c1 = pl.ds((g + 1) * L, L)
                a0 = rows[rb, c0] * w0
                b0 = rows[rb, c1] * w0
                a1 = rows[rb + 1, c0] * w1
                b1 = rows[rb + 1, c1] * w1
                a2 = rows[rb + 2, c0] * w2
                b2 = rows[rb + 2, c1] * w2
                a3 = rows[rb + 3, c0] * w3
                b3 = rows[rb + 3, c1] * w3
                acc_a = (a0 + a1) + (a2 + a3)
                acc_b = (b0 + b1) + (b2 + b3)
                plsc.store_scatter(ob, [iota49 + (g * L * P + p)], acc_a)
                plsc.store_scatter(ob, [iota49 + ((g + 1) * L * P + p)], acc_b)

    def out_slice(r_glob):
        return out_hbm.at[pl.ds(r_glob * (C * P), C * P)]

    def one_roi(j, ob, so):
        r_glob = rbase + j

        @pl.when(jnp.logical_and(j >= 2, r_glob - 2 < R))
        def _wait_prev_out():
            pltpu.make_async_copy(ob, out_slice(0), so).wait()

        wait_a(j)
        issue_b(j)
        accumulate(j, rows_a, ob, 0, PA, 0)
        wait_b(j)

        @pl.when(j + 1 < RPW)
        def _prefetch_next():
            issue_a(j + 1)

        accumulate(j, rows_b, ob, PA, P, PA * 4)

        @pl.when(r_glob < R)
        def _write_out():
            pltpu.async_copy(ob, out_slice(r_glob), so)

    issue_a(0)

    @pl.loop(0, RPW, step=2)
    def _roi_pair(j):
        one_roi(j, ob0, so0)
        one_roi(j + 1, ob1, so1)

    @pl.when(rbase + RPW - 2 < R)
    def _drain0():
        pltpu.make_async_copy(ob0, out_slice(0), so0).wait()

    @pl.when(rbase + RPW - 1 < R)
    def _drain1():
        pltpu.make_async_copy(ob1, out_slice(0), so1).wait()


def kernel(features, rois):
    table = features.transpose(0, 2, 3, 1).reshape(B * H * W, C)
    out_flat = _roi_sample_sc(table, rois.reshape(-1))
    return out_flat.reshape(R, C, PH, PW)


# speedup vs baseline: 12.7968x; 3.1436x over previous
import functools

import jax
import jax.numpy as jnp
from jax import lax
from jax.experimental import pallas as pl
from jax.experimental.pallas import tpu as pltpu
from jax.experimental.pallas import tpu_sc as plsc

B, C, H, W = 2, 256, 128, 128
R = 1000
PH, PW = 7, 7
P = PH * PW
SPATIAL_SCALE = 0.25

NC, NS, L = 2, 16, 16
NW = NC * NS
S_PAD = 49152
SPW = S_PAD // NW
CS = 32
NCHUNK = SPW // CS
NGRP = SPW // L

_mesh = plsc.VectorSubcoreMesh(core_axis_name="c", subcore_axis_name="s")


@functools.partial(
    pl.kernel,
    out_type=jax.ShapeDtypeStruct((S_PAD * C,), jnp.float32),
    mesh=_mesh,
    compiler_params=pltpu.CompilerParams(needs_layout_passes=False),
    scratch_types=[
        pltpu.VMEM((R * 5,), jnp.float32),
        pltpu.VMEM((SPW * 4,), jnp.int32),
        pltpu.VMEM((SPW * 4,), jnp.float32),
        pltpu.VMEM((CS * 4, C), jnp.float32),
        pltpu.VMEM((CS * 4, C), jnp.float32),
        pltpu.VMEM((CS * C,), jnp.float32),
        pltpu.VMEM((CS * C,), jnp.float32),
        pltpu.SemaphoreType.DMA,
        pltpu.SemaphoreType.DMA,
        pltpu.SemaphoreType.DMA,
        pltpu.SemaphoreType.DMA,
    ],
)
def _roi_sample_sc(table_hbm, rois_hbm, out_hbm, rois_v, idx_v, wts_v,
                   rb0, rb1, ob0, ob1, sg0, sg1, so0, so1):
    wid = lax.axis_index("s") * NC + lax.axis_index("c")
    wbase = wid * SPW

    pltpu.sync_copy(rois_hbm, rois_v)

    iota = lax.iota(jnp.int32, L)
    zero16 = jnp.zeros((L,), jnp.int32)

    @pl.loop(0, NGRP)
    def _compute_indices(g):
        s_glob = wbase + g * L + iota
        r_raw = s_glob // P
        p = s_glob - r_raw * P
        r = jnp.minimum(r_raw, R - 1)
        r5 = r * 5
        b = plsc.load_gather(rois_v, [r5]).astype(jnp.int32)
        x1 = plsc.load_gather(rois_v, [r5 + 1])
        y1 = plsc.load_gather(rois_v, [r5 + 2])
        x2 = plsc.load_gather(rois_v, [r5 + 3])
        y2 = plsc.load_gather(rois_v, [r5 + 4])
        relx = (p % PW).astype(jnp.float32) * (1.0 / PW) + (0.5 / PW)
        rely = (p // PW).astype(jnp.float32) * (1.0 / PH) + (0.5 / PH)
        px = (x1 + relx * (x2 - x1)) * SPATIAL_SCALE - 0.5
        py = (y1 + rely * (y2 - y1)) * SPATIAL_SCALE - 0.5
        x0 = (px + 1.0).astype(jnp.int32) - 1
        y0 = (py + 1.0).astype(jnp.int32) - 1
        wx1 = px - x0.astype(jnp.float32)
        wx0 = 1.0 - wx1
        wy1 = py - y0.astype(jnp.float32)
        wy0 = 1.0 - wy1
        vx0 = jnp.where(x0 >= 0, 1.0, 0.0)
        vx1 = jnp.where(x0 + 1 <= W - 1, 1.0, 0.0)
        vy0 = jnp.where(y0 >= 0, 1.0, 0.0)
        vy1 = jnp.where(y0 + 1 <= H - 1, 1.0, 0.0)
        xc0 = jnp.clip(x0, 0, W - 1)
        xc1 = jnp.clip(x0 + 1, 0, W - 1)
        yc0 = jnp.clip(y0, 0, H - 1)
        yc1 = jnp.clip(y0 + 1, 0, H - 1)
        base = b * (H * W)
        row0 = base + yc0 * W
        row1 = base + yc1 * W
        pos = iota * 4 + g * (4 * L)
        plsc.store_scatter(idx_v, [pos], row0 + xc0)
        plsc.store_scatter(idx_v, [pos + 1], row0 + xc1)
        plsc.store_scatter(idx_v, [pos + 2], row1 + xc0)
        plsc.store_scatter(idx_v, [pos + 3], row1 + xc1)
        plsc.store_scatter(wts_v, [pos], wy0 * wx0 * vy0 * vx0)
        plsc.store_scatter(wts_v, [pos + 1], wy0 * wx1 * vy0 * vx1)
        plsc.store_scatter(wts_v, [pos + 2], wy1 * wx0 * vy1 * vx0)
        plsc.store_scatter(wts_v, [pos + 3], wy1 * wx1 * vy1 * vx1)

    def issue_gather(c, rb, sg):
        pltpu.async_copy(
            table_hbm.at[idx_v.at[pl.ds(c * (CS * 4), CS * 4)]], rb, sg)

    def wait_gather(c, rb, sg):
        pltpu.make_async_copy(
            table_hbm.at[idx_v.at[pl.ds(c * (CS * 4), CS * 4)]], rb, sg
        ).wait()

    def out_slice(c):
        return out_hbm.at[pl.ds((wbase + c * CS) * C, CS * C)]

    def compute(c, rb, ob):
        @pl.loop(0, CS)
        def _point(s):
            k4 = c * (CS * 4) + s * 4
            w0 = plsc.load_gather(wts_v, [zero16 + k4])
            w1 = plsc.load_gather(wts_v, [zero16 + (k4 + 1)])
            w2 = plsc.load_gather(wts_v, [zero16 + (k4 + 2)])
            w3 = plsc.load_gather(wts_v, [zero16 + (k4 + 3)])
            rbs = s * 4
            for g in range(0, C // L, 2):
                c0 = pl.ds(g * L, L)
                c1 = pl.ds((g + 1) * L, L)
                a0 = rb[rbs, c0] * w0
                b0 = rb[rbs, c1] * w0
                a1 = rb[rbs + 1, c0] * w1
                b1 = rb[rbs + 1, c1] * w1
                a2 = rb[rbs + 2, c0] * w2
                b2 = rb[rbs + 2, c1] * w2
                a3 = rb[rbs + 3, c0] * w3
                b3 = rb[rbs + 3, c1] * w3
                ob[pl.ds(s * C + g * L, L)] = (a0 + a1) + (a2 + a3)
                ob[pl.ds(s * C + (g + 1) * L, L)] = (b0 + b1) + (b2 + b3)

    def half(c, rb, sg, ob, so, rb_next, sg_next):
        @pl.when(c >= 2)
        def _wait_prev_out():
            pltpu.make_async_copy(ob, out_slice(0), so).wait()

        @pl.when(c + 1 < NCHUNK)
        def _prefetch_next():
            issue_gather(c + 1, rb_next, sg_next)

        wait_gather(c, rb, sg)
        compute(c, rb, ob)
        pltpu.async_copy(ob, out_slice(c), so)

    issue_gather(0, rb0, sg0)

    @pl.loop(0, NCHUNK, step=2)
    def _chunk_pair(c):
        half(c, rb0, sg0, ob0, so0, rb1, sg1)
        half(c + 1, rb1, sg1, ob1, so1, rb0, sg0)

    pltpu.make_async_copy(ob0, out_slice(0), so0).wait()
    pltpu.make_async_copy(ob1, out_slice(0), so1).wait()


def kernel(features, rois):
    table = features.transpose(0, 2, 3, 1).reshape(B * H * W, C)
    out_flat = _roi_sample_sc(table, rois.reshape(-1))
    out = out_flat.reshape(S_PAD, C)[: R * P]
    return out.reshape(R, P, C).transpose(0, 2, 1).reshape(R, C, PH, PW)


# speedup vs baseline: 13.7703x; 1.0761x over previous
import functools

import jax
import jax.numpy as jnp
from jax import lax
from jax.experimental import pallas as pl
from jax.experimental.pallas import tpu as pltpu
from jax.experimental.pallas import tpu_sc as plsc

B, C, H, W = 2, 256, 128, 128
R = 1000
PH, PW = 7, 7
P = PH * PW
SPATIAL_SCALE = 0.25

NC, NS, L = 2, 16, 16
NW = NC * NS
S_PAD = 49152
SPW = S_PAD // NW
CS = 32
NCHUNK = SPW // CS
NGRP = SPW // L

_mesh = plsc.VectorSubcoreMesh(core_axis_name="c", subcore_axis_name="s")


@functools.partial(
    pl.kernel,
    out_type=jax.ShapeDtypeStruct((S_PAD * C,), jnp.float32),
    mesh=_mesh,
    compiler_params=pltpu.CompilerParams(needs_layout_passes=False),
    scratch_types=[
        pltpu.VMEM((R * 5,), jnp.float32),
        pltpu.VMEM((SPW * 4,), jnp.int32),
        pltpu.VMEM((SPW * 4,), jnp.float32),
        pltpu.VMEM((CS * 4, C), jnp.float32),
        pltpu.VMEM((CS * 4, C), jnp.float32),
        pltpu.VMEM((CS * C,), jnp.float32),
        pltpu.VMEM((CS * C,), jnp.float32),
        pltpu.SemaphoreType.DMA,
        pltpu.SemaphoreType.DMA,
        pltpu.SemaphoreType.DMA,
        pltpu.SemaphoreType.DMA,
    ],
)
def _roi_sample_sc(table_hbm, rois_hbm, out_hbm, rois_v, idx_v, wts_v,
                   rb0, rb1, ob0, ob1, sg0, sg1, so0, so1):
    wid = lax.axis_index("s") * NC + lax.axis_index("c")
    wbase = wid * SPW

    pltpu.sync_copy(rois_hbm, rois_v)

    iota = lax.iota(jnp.int32, L)
    zero16 = jnp.zeros((L,), jnp.int32)

    @pl.loop(0, NGRP)
    def _compute_indices(g):
        s_glob = wbase + g * L + iota
        r_raw = s_glob // P
        p = s_glob - r_raw * P
        r = jnp.minimum(r_raw, R - 1)
        r5 = r * 5
        b = plsc.load_gather(rois_v, [r5]).astype(jnp.int32)
        x1 = plsc.load_gather(rois_v, [r5 + 1])
        y1 = plsc.load_gather(rois_v, [r5 + 2])
        x2 = plsc.load_gather(rois_v, [r5 + 3])
        y2 = plsc.load_gather(rois_v, [r5 + 4])
        relx = (p % PW).astype(jnp.float32) * (1.0 / PW) + (0.5 / PW)
        rely = (p // PW).astype(jnp.float32) * (1.0 / PH) + (0.5 / PH)
        px = (x1 + relx * (x2 - x1)) * SPATIAL_SCALE - 0.5
        py = (y1 + rely * (y2 - y1)) * SPATIAL_SCALE - 0.5
        x0 = (px + 1.0).astype(jnp.int32) - 1
        y0 = (py + 1.0).astype(jnp.int32) - 1
        wx1 = px - x0.astype(jnp.float32)
        wx0 = 1.0 - wx1
        wy1 = py - y0.astype(jnp.float32)
        wy0 = 1.0 - wy1
        vx0 = jnp.where(x0 >= 0, 1.0, 0.0)
        vx1 = jnp.where(x0 + 1 <= W - 1, 1.0, 0.0)
        vy0 = jnp.where(y0 >= 0, 1.0, 0.0)
        vy1 = jnp.where(y0 + 1 <= H - 1, 1.0, 0.0)
        xc0 = jnp.clip(x0, 0, W - 1)
        xc1 = jnp.clip(x0 + 1, 0, W - 1)
        yc0 = jnp.clip(y0, 0, H - 1)
        yc1 = jnp.clip(y0 + 1, 0, H - 1)
        base = b * (H * W)
        row0 = base + yc0 * W
        row1 = base + yc1 * W
        pos = iota * 4 + g * (4 * L)
        plsc.store_scatter(idx_v, [pos], row0 + xc0)
        plsc.store_scatter(idx_v, [pos + 1], row0 + xc1)
        plsc.store_scatter(idx_v, [pos + 2], row1 + xc0)
        plsc.store_scatter(idx_v, [pos + 3], row1 + xc1)
        plsc.store_scatter(wts_v, [pos], wy0 * wx0 * vy0 * vx0)
        plsc.store_scatter(wts_v, [pos + 1], wy0 * wx1 * vy0 * vx1)
        plsc.store_scatter(wts_v, [pos + 2], wy1 * wx0 * vy1 * vx0)
        plsc.store_scatter(wts_v, [pos + 3], wy1 * wx1 * vy1 * vx1)

    def issue_gather(c, rb, sg):
        pltpu.async_copy(
            table_hbm.at[idx_v.at[pl.ds(c * (CS * 4), CS * 4)]], rb, sg)

    def wait_gather(c, rb, sg):
        pltpu.make_async_copy(
            table_hbm.at[idx_v.at[pl.ds(c * (CS * 4), CS * 4)]], rb, sg
        ).wait()

    def out_slice(c):
        return out_hbm.at[pl.ds((wbase + c * CS) * C, CS * C)]

    def compute(c, rb, ob):
        @pl.loop(0, CS)
        def _point(s):
            k4 = c * (CS * 4) + s * 4
            w0 = plsc.load_gather(wts_v, [zero16 + k4])
            w1 = plsc.load_gather(wts_v, [zero16 + (k4 + 1)])
            w2 = plsc.load_gather(wts_v, [zero16 + (k4 + 2)])
            w3 = plsc.load_gather(wts_v, [zero16 + (k4 + 3)])
            rbs = s * 4
            for g in range(0, C // L, 4):
                accs = []
                for q in range(4):
                    cq = pl.ds((g + q) * L, L)
                    t0 = rb[rbs, cq] * w0
                    t1 = rb[rbs + 1, cq] * w1
                    t2 = rb[rbs + 2, cq] * w2
                    t3 = rb[rbs + 3, cq] * w3
                    accs.append((t0 + t1) + (t2 + t3))
                for q in range(4):
                    ob[pl.ds(s * C + (g + q) * L, L)] = accs[q]

    def half(c, rb, sg, ob, so, rb_next, sg_next):
        @pl.when(c >= 2)
        def _wait_prev_out():
            pltpu.make_async_copy(ob, out_slice(0), so).wait()

        @pl.when(c + 1 < NCHUNK)
        def _prefetch_next():
            issue_gather(c + 1, rb_next, sg_next)

        wait_gather(c, rb, sg)
        compute(c, rb, ob)
        pltpu.async_copy(ob, out_slice(c), so)

    issue_gather(0, rb0, sg0)

    @pl.loop(0, NCHUNK, step=2)
    def _chunk_pair(c):
        half(c, rb0, sg0, ob0, so0, rb1, sg1)
        half(c + 1, rb1, sg1, ob1, so1, rb0, sg0)

    pltpu.make_async_copy(ob0, out_slice(0), so0).wait()
    pltpu.make_async_copy(ob1, out_slice(0), so1).wait()


def kernel(features, rois):
    table = features.transpose(0, 2, 3, 1).reshape(B * H * W, C)
    out_flat = _roi_sample_sc(table, rois.reshape(-1))
    out = out_flat.reshape(S_PAD, C)[: R * P]
    return out.reshape(R, P, C).transpose(0, 2, 1).reshape(R, C, PH, PW)
